# baseline (speedup 1.0000x reference)
import functools

import jax
import jax.numpy as jnp
from jax import lax
from jax.experimental import pallas as pl
from jax.experimental.pallas import tpu as pltpu
from jax.experimental.pallas import tpu_sc as plsc

_NC, _NS = 2, 16
_NW = _NC * _NS
_CH = 200
_SUB = 200


def _sc_gather(table, idx_flat):
    e_total = idx_flat.shape[0]
    f = table.shape[1]
    dt = table.dtype
    rows_per_w = e_total // _NW
    n_ch = rows_per_w // _CH
    n_sub = _CH // _SUB
    mesh = plsc.VectorSubcoreMesh(core_axis_name="c", subcore_axis_name="s",
                                  num_cores=_NC, num_subcores=_NS)

    @functools.partial(
        pl.kernel, mesh=mesh,
        out_type=jax.ShapeDtypeStruct((e_total, f), dt),
        scratch_types=[
            pltpu.VMEM((_CH,), jnp.int32),
            pltpu.VMEM((_CH,), jnp.int32),
            pltpu.VMEM((_CH, f), dt),
            pltpu.VMEM((_CH, f), dt),
            pltpu.SemaphoreType.DMA,
            pltpu.SemaphoreType.DMA,
            pltpu.SemaphoreType.DMA,
        ],
    )
    def k(table_hbm, idx_hbm, out_hbm, idx_v0, idx_v1, rows_v0, rows_v1,
          isem, gsem, osem):
        wid = lax.axis_index("s") * _NC + lax.axis_index("c")
        base = wid * rows_per_w
        idx_vs = (idx_v0, idx_v1)
        rows_vs = (rows_v0, rows_v1)

        def idx_cp(i, slot):
            return pltpu.make_async_copy(
                idx_hbm.at[pl.ds(base + i * _CH, _CH)], idx_vs[slot], isem)

        def gather_cps(slot):
            return [pltpu.make_async_copy(
                table_hbm.at[idx_vs[slot].at[pl.ds(j * _SUB, _SUB)]],
                rows_vs[slot].at[pl.ds(j * _SUB, _SUB)], gsem)
                for j in range(n_sub)]

        def out_cp(i, slot):
            return pltpu.make_async_copy(
                rows_vs[slot], out_hbm.at[pl.ds(base + i * _CH, _CH)], osem)

        def body(i, carry):
            idx_cp(i, 0).start()
            idx_cp(i, 0).wait()
            copies = gather_cps(0)
            for cp in copies:
                cp.start()
            for cp in copies:
                cp.wait()
            out_cp(i, 0).start()
            out_cp(i, 0).wait()
            return carry

        lax.fori_loop(0, n_ch, body, 0)

    return k(table, idx_flat)


def _edge_preact(ag, a_blk, b3, ws, wn, wb, b, nb, m):
    two_f = ws.shape[1]
    bf = jnp.bfloat16
    s = jnp.dot(a_blk.astype(bf), ws, preferred_element_type=jnp.float32)
    ynb = jnp.dot(ag.astype(bf), wn, preferred_element_type=jnp.float32)
    q = jnp.dot(b3, wb, preferred_element_type=jnp.float32)
    s_exp = jnp.broadcast_to(s[:, None, :], (nb, m, two_f)).reshape(nb * m, two_f)
    return ynb + q + s_exp + b


def _stats_body(nb, m, ag_ref, a_ref, b3_ref, ws_ref, wn_ref, wb_ref,
                b_ref, out_ref):
    y = _edge_preact(ag_ref[...], a_ref[...], b3_ref[...],
                     ws_ref[...], wn_ref[...], wb_ref[...], b_ref[...], nb, m)

    @pl.when(pl.program_id(0) == 0)
    def _():
        out_ref[...] = jnp.zeros_like(out_ref)

    ssum = jnp.sum(y, axis=0, keepdims=True)
    ssq = jnp.sum(y * y, axis=0, keepdims=True)
    out_ref[...] += jnp.concatenate([ssum, ssq], axis=0)


def _main_body(nb, m, e_total, stats_ref, ag_ref, a_ref, b3_ref, ws_ref,
               wn_ref, wb_ref, b_ref, g1_ref, b1_ref, out_ref):
    y = _edge_preact(ag_ref[...], a_ref[...], b3_ref[...],
                     ws_ref[...], wn_ref[...], wb_ref[...], b_ref[...], nb, m)
    stats = stats_ref[...]
    mean = stats[0:1, :] / e_total
    var = stats[1:2, :] / e_total - mean * mean
    inv = lax.rsqrt(var + 1e-5)
    z = (y - mean) * (inv * g1_ref[...]) + b1_ref[...]
    f = z.shape[1] // 2
    filt = jax.nn.sigmoid(z[:, :f])
    core = jax.nn.softplus(z[:, f:])
    p = (filt * core).reshape(nb, m, f)
    out_ref[...] = jnp.sum(p, axis=1)


def _final_body(a_ref, *rest):
    ps_refs = rest[:-3]
    g2_ref, b2_ref, out_ref = rest[-3:]
    x = jnp.concatenate([p[...] for p in ps_refs], axis=0)
    mean = jnp.mean(x, axis=0, keepdims=True)
    d = x - mean
    var = jnp.mean(d * d, axis=0, keepdims=True)
    z = d * lax.rsqrt(var + 1e-5) * g2_ref[...] + b2_ref[...]
    out_ref[...] = jax.nn.softplus(a_ref[...] + z)


_NB = 200


def _stats_call(ag, a_k, bf_k, ws, wn, wb, b, interpret=False):
    nk, f = a_k.shape
    e_k, hp = ag.shape
    m = e_k // nk
    two_f = ws.shape[1]
    de = bf_k.shape[1]
    r = _NB * m
    full = lambda shape: pl.BlockSpec(shape, lambda i: (0,) * len(shape))
    return pl.pallas_call(
        functools.partial(_stats_body, _NB, m),
        grid=(nk // _NB,),
        in_specs=[
            pl.BlockSpec((r, hp), lambda i: (i, 0)),
            pl.BlockSpec((_NB, f), lambda i: (i, 0)),
            pl.BlockSpec((r, de), lambda i: (i, 0)),
            full((f, two_f)), full((f, two_f)), full((de, two_f)),
            full((1, two_f)),
        ],
        out_specs=pl.BlockSpec((2, two_f), lambda i: (0, 0)),
        out_shape=jax.ShapeDtypeStruct((2, two_f), jnp.float32),
        interpret=interpret,
    )(ag, a_k, bf_k, ws, wn, wb, b)


def _main_call(stats, ag, a_k, bf_k, ws, wn, wb, b, g1, b1, e_total,
               interpret=False):
    nk, f = a_k.shape
    e_k, hp = ag.shape
    m = e_k // nk
    two_f = ws.shape[1]
    de = bf_k.shape[1]
    r = _NB * m
    full = lambda shape: pl.BlockSpec(shape, lambda i: (0,) * len(shape))
    return pl.pallas_call(
        functools.partial(_main_body, _NB, m, float(e_total)),
        grid=(nk // _NB,),
        in_specs=[
            full((2, two_f)),
            pl.BlockSpec((r, hp), lambda i: (i, 0)),
            pl.BlockSpec((_NB, f), lambda i: (i, 0)),
            pl.BlockSpec((r, de), lambda i: (i, 0)),
            full((f, two_f)), full((f, two_f)), full((de, two_f)),
            full((1, two_f)), full((1, two_f)), full((1, two_f)),
        ],
        out_specs=pl.BlockSpec((_NB, f), lambda i: (i, 0)),
        out_shape=jax.ShapeDtypeStruct((nk, f), jnp.float32),
        interpret=interpret,
    )(stats, ag, a_k, bf_k, ws, wn, wb, b, g1, b1)


_K = 2


def kernel(atom_features, bond_features, W, b, bn1_scale, bn1_bias,
           bn2_scale, bn2_bias, neighbor_indices):
    a = atom_features
    n, f = a.shape
    m = neighbor_indices.shape[1]
    idx_flat = neighbor_indices.reshape(-1)
    de = bond_features.shape[2]
    bflat = bond_features.reshape(n * m, de)
    w16 = W.astype(jnp.bfloat16)
    ws, wn = w16[:f], w16[f:2 * f]
    wb = W[2 * f:]
    b1r = b.reshape(1, -1)
    g1 = bn1_scale.reshape(1, -1)
    bb1 = bn1_bias.reshape(1, -1)
    g2 = bn2_scale.reshape(1, -1)
    bb2 = bn2_bias.reshape(1, -1)

    e_total = n * m
    ek, nk = e_total // _K, n // _K
    ags, sts = [], []
    for k in range(_K):
        ags.append(_sc_gather(a, idx_flat[k * ek:(k + 1) * ek]))
        sts.append(_stats_call(ags[k], a[k * nk:(k + 1) * nk],
                               bflat[k * ek:(k + 1) * ek], ws, wn, wb, b1r))
    stats = sts[0]
    for st in sts[1:]:
        stats = stats + st
    presums = [
        _main_call(stats, ags[k], a[k * nk:(k + 1) * nk],
                   bflat[k * ek:(k + 1) * ek], ws, wn, wb, b1r, g1, bb1,
                   e_total)
        for k in range(_K)
    ]
    out = pl.pallas_call(
        _final_body,
        out_shape=jax.ShapeDtypeStruct((n, f), jnp.float32),
    )(a, *presums, g2, bb2)
    return out


# speedup vs baseline: 1.1273x; 1.1273x over previous
import functools

import jax
import jax.numpy as jnp
from jax import lax
from jax.experimental import pallas as pl
from jax.experimental.pallas import tpu as pltpu
from jax.experimental.pallas import tpu_sc as plsc

_NC, _NS = 2, 16
_NW = _NC * _NS
_CH = 400
_SUB = 400


def _sc_gather(table, idx_flat):
    e_total = idx_flat.shape[0]
    f = table.shape[1]
    dt = table.dtype
    rows_per_w = e_total // _NW
    n_ch = rows_per_w // _CH
    n_sub = _CH // _SUB
    mesh = plsc.VectorSubcoreMesh(core_axis_name="c", subcore_axis_name="s",
                                  num_cores=_NC, num_subcores=_NS)

    @functools.partial(
        pl.kernel, mesh=mesh,
        out_type=jax.ShapeDtypeStruct((e_total, f), dt),
        scratch_types=[
            pltpu.VMEM((_CH,), jnp.int32),
            pltpu.VMEM((_CH,), jnp.int32),
            pltpu.VMEM((_CH, f), dt),
            pltpu.VMEM((_CH, f), dt),
            pltpu.SemaphoreType.DMA,
            pltpu.SemaphoreType.DMA,
            pltpu.SemaphoreType.DMA,
        ],
    )
    def k(table_hbm, idx_hbm, out_hbm, idx_v0, idx_v1, rows_v0, rows_v1,
          isem, gsem, osem):
        wid = lax.axis_index("s") * _NC + lax.axis_index("c")
        base = wid * rows_per_w
        idx_vs = (idx_v0, idx_v1)
        rows_vs = (rows_v0, rows_v1)

        def idx_cp(i, slot):
            return pltpu.make_async_copy(
                idx_hbm.at[pl.ds(base + i * _CH, _CH)], idx_vs[slot], isem)

        def gather_cps(slot):
            return [pltpu.make_async_copy(
                table_hbm.at[idx_vs[slot].at[pl.ds(j * _SUB, _SUB)]],
                rows_vs[slot].at[pl.ds(j * _SUB, _SUB)], gsem)
                for j in range(n_sub)]

        def out_cp(i, slot):
            return pltpu.make_async_copy(
                rows_vs[slot], out_hbm.at[pl.ds(base + i * _CH, _CH)], osem)

        def body(i, carry):
            idx_cp(i, 0).start()
            idx_cp(i, 0).wait()
            copies = gather_cps(0)
            for cp in copies:
                cp.start()
            for cp in copies:
                cp.wait()
            out_cp(i, 0).start()
            out_cp(i, 0).wait()
            return carry

        lax.fori_loop(0, n_ch, body, 0)

    return k(table, idx_flat)


def _edge_preact(ag, a_blk, b3, ws, wn, wb, b, nb, m):
    two_f = ws.shape[1]
    bf = jnp.bfloat16
    s = jnp.dot(a_blk.astype(bf), ws, preferred_element_type=jnp.float32)
    ynb = jnp.dot(ag.astype(bf), wn, preferred_element_type=jnp.float32)
    q = jnp.dot(b3, wb, preferred_element_type=jnp.float32)
    s_exp = jnp.broadcast_to(s[:, None, :], (nb, m, two_f)).reshape(nb * m, two_f)
    return ynb + q + s_exp + b


def _stats_body(nb, m, ag_ref, a_ref, b3_ref, ws_ref, wn_ref, wb_ref,
                b_ref, out_ref):
    y = _edge_preact(ag_ref[...], a_ref[...], b3_ref[...],
                     ws_ref[...], wn_ref[...], wb_ref[...], b_ref[...], nb, m)

    @pl.when(pl.program_id(0) == 0)
    def _():
        out_ref[...] = jnp.zeros_like(out_ref)

    ssum = jnp.sum(y, axis=0, keepdims=True)
    ssq = jnp.sum(y * y, axis=0, keepdims=True)
    out_ref[...] += jnp.concatenate([ssum, ssq], axis=0)


def _main_body(nb, m, e_total, stats_ref, ag_ref, a_ref, b3_ref, ws_ref,
               wn_ref, wb_ref, b_ref, g1_ref, b1_ref, out_ref):
    y = _edge_preact(ag_ref[...], a_ref[...], b3_ref[...],
                     ws_ref[...], wn_ref[...], wb_ref[...], b_ref[...], nb, m)
    stats = stats_ref[...]
    mean = stats[0:1, :] / e_total
    var = stats[1:2, :] / e_total - mean * mean
    inv = lax.rsqrt(var + 1e-5)
    z = (y - mean) * (inv * g1_ref[...]) + b1_ref[...]
    f = z.shape[1] // 2
    filt = jax.nn.sigmoid(z[:, :f])
    core = jax.nn.softplus(z[:, f:])
    p = (filt * core).reshape(nb, m, f)
    out_ref[...] = jnp.sum(p, axis=1)


def _final_body(a_ref, *rest):
    ps_refs = rest[:-3]
    g2_ref, b2_ref, out_ref = rest[-3:]
    x = jnp.concatenate([p[...] for p in ps_refs], axis=0)
    mean = jnp.mean(x, axis=0, keepdims=True)
    d = x - mean
    var = jnp.mean(d * d, axis=0, keepdims=True)
    z = d * lax.rsqrt(var + 1e-5) * g2_ref[...] + b2_ref[...]
    out_ref[...] = jax.nn.softplus(a_ref[...] + z)


_NB = 200


def _stats_call(ag, a_k, bf_k, ws, wn, wb, b, interpret=False):
    nk, f = a_k.shape
    e_k, hp = ag.shape
    m = e_k // nk
    two_f = ws.shape[1]
    de = bf_k.shape[1]
    r = _NB * m
    full = lambda shape: pl.BlockSpec(shape, lambda i: (0,) * len(shape))
    return pl.pallas_call(
        functools.partial(_stats_body, _NB, m),
        grid=(nk // _NB,),
        in_specs=[
            pl.BlockSpec((r, hp), lambda i: (i, 0)),
            pl.BlockSpec((_NB, f), lambda i: (i, 0)),
            pl.BlockSpec((r, de), lambda i: (i, 0)),
            full((f, two_f)), full((f, two_f)), full((de, two_f)),
            full((1, two_f)),
        ],
        out_specs=pl.BlockSpec((2, two_f), lambda i: (0, 0)),
        out_shape=jax.ShapeDtypeStruct((2, two_f), jnp.float32),
        interpret=interpret,
    )(ag, a_k, bf_k, ws, wn, wb, b)


def _main_call(stats, ag, a_k, bf_k, ws, wn, wb, b, g1, b1, e_total,
               interpret=False):
    nk, f = a_k.shape
    e_k, hp = ag.shape
    m = e_k // nk
    two_f = ws.shape[1]
    de = bf_k.shape[1]
    r = _NB * m
    full = lambda shape: pl.BlockSpec(shape, lambda i: (0,) * len(shape))
    return pl.pallas_call(
        functools.partial(_main_body, _NB, m, float(e_total)),
        grid=(nk // _NB,),
        in_specs=[
            full((2, two_f)),
            pl.BlockSpec((r, hp), lambda i: (i, 0)),
            pl.BlockSpec((_NB, f), lambda i: (i, 0)),
            pl.BlockSpec((r, de), lambda i: (i, 0)),
            full((f, two_f)), full((f, two_f)), full((de, two_f)),
            full((1, two_f)), full((1, two_f)), full((1, two_f)),
        ],
        out_specs=pl.BlockSpec((_NB, f), lambda i: (i, 0)),
        out_shape=jax.ShapeDtypeStruct((nk, f), jnp.float32),
        interpret=interpret,
    )(stats, ag, a_k, bf_k, ws, wn, wb, b, g1, b1)


_K = 1


def kernel(atom_features, bond_features, W, b, bn1_scale, bn1_bias,
           bn2_scale, bn2_bias, neighbor_indices):
    a = atom_features
    n, f = a.shape
    m = neighbor_indices.shape[1]
    idx_flat = neighbor_indices.reshape(-1)
    de = bond_features.shape[2]
    bflat = bond_features.reshape(n * m, de).astype(jnp.bfloat16)
    w16 = W.astype(jnp.bfloat16)
    ws, wn, wb = w16[:f], w16[f:2 * f], w16[2 * f:]
    b1r = b.reshape(1, -1)
    g1 = bn1_scale.reshape(1, -1)
    bb1 = bn1_bias.reshape(1, -1)
    g2 = bn2_scale.reshape(1, -1)
    bb2 = bn2_bias.reshape(1, -1)

    e_total = n * m
    ek, nk = e_total // _K, n // _K
    ags, sts = [], []
    for k in range(_K):
        ags.append(_sc_gather(a, idx_flat[k * ek:(k + 1) * ek]))
        sts.append(_stats_call(ags[k], a[k * nk:(k + 1) * nk],
                               bflat[k * ek:(k + 1) * ek], ws, wn, wb, b1r))
    stats = sts[0]
    for st in sts[1:]:
        stats = stats + st
    presums = [
        _main_call(stats, ags[k], a[k * nk:(k + 1) * nk],
                   bflat[k * ek:(k + 1) * ek], ws, wn, wb, b1r, g1, bb1,
                   e_total)
        for k in range(_K)
    ]
    out = pl.pallas_call(
        _final_body,
        out_shape=jax.ShapeDtypeStruct((n, f), jnp.float32),
    )(a, *presums, g2, bb2)
    return out


# speedup vs baseline: 1.1458x; 1.0165x over previous
import functools

import jax
import jax.numpy as jnp
from jax import lax
from jax.experimental import pallas as pl
from jax.experimental.pallas import tpu as pltpu
from jax.experimental.pallas import tpu_sc as plsc

_NC, _NS = 2, 16
_NW = _NC * _NS
_CH = 400
_SUB = 400


def _sc_gather(table, idx_flat):
    e_total = idx_flat.shape[0]
    f = table.shape[1]
    dt = table.dtype
    rows_per_w = e_total // _NW
    n_ch = rows_per_w // _CH
    n_sub = _CH // _SUB
    mesh = plsc.VectorSubcoreMesh(core_axis_name="c", subcore_axis_name="s",
                                  num_cores=_NC, num_subcores=_NS)

    @functools.partial(
        pl.kernel, mesh=mesh,
        out_type=jax.ShapeDtypeStruct((e_total, f), dt),
        scratch_types=[
            pltpu.VMEM((_CH,), jnp.int32),
            pltpu.VMEM((_CH,), jnp.int32),
            pltpu.VMEM((_CH, f), dt),
            pltpu.VMEM((_CH, f), dt),
            pltpu.SemaphoreType.DMA,
            pltpu.SemaphoreType.DMA,
            pltpu.SemaphoreType.DMA,
        ],
    )
    def k(table_hbm, idx_hbm, out_hbm, idx_v0, idx_v1, rows_v0, rows_v1,
          isem, gsem, osem):
        wid = lax.axis_index("s") * _NC + lax.axis_index("c")
        base = wid * rows_per_w
        idx_vs = (idx_v0, idx_v1)
        rows_vs = (rows_v0, rows_v1)

        def idx_cp(i, slot):
            return pltpu.make_async_copy(
                idx_hbm.at[pl.ds(base + i * _CH, _CH)], idx_vs[slot], isem)

        def gather_cps(slot):
            return [pltpu.make_async_copy(
                table_hbm.at[idx_vs[slot].at[pl.ds(j * _SUB, _SUB)]],
                rows_vs[slot].at[pl.ds(j * _SUB, _SUB)], gsem)
                for j in range(n_sub)]

        def out_cp(i, slot):
            return pltpu.make_async_copy(
                rows_vs[slot], out_hbm.at[pl.ds(base + i * _CH, _CH)], osem)

        def body(i, carry):
            idx_cp(i, 0).start()
            idx_cp(i, 0).wait()
            copies = gather_cps(0)
            for cp in copies:
                cp.start()
            for cp in copies:
                cp.wait()
            out_cp(i, 0).start()
            out_cp(i, 0).wait()
            return carry

        lax.fori_loop(0, n_ch, body, 0)

    return k(table, idx_flat)


def _edge_preact(ag, a_blk, b3, ws, wn, wb, b, nb, m):
    two_f = ws.shape[1]
    bf = jnp.bfloat16
    s = jnp.dot(a_blk.astype(bf), ws, preferred_element_type=jnp.float32)
    ynb = jnp.dot(ag.astype(bf), wn, preferred_element_type=jnp.float32)
    q = jnp.dot(b3, wb, preferred_element_type=jnp.float32)
    s_exp = jnp.broadcast_to(s[:, None, :], (nb, m, two_f)).reshape(nb * m, two_f)
    return ynb + q + s_exp + b


def _stats_body(nb, m, ag_ref, a_ref, b3_ref, ws_ref, wn_ref, wb_ref,
                b_ref, out_ref):
    y = _edge_preact(ag_ref[...], a_ref[...], b3_ref[...],
                     ws_ref[...], wn_ref[...], wb_ref[...], b_ref[...], nb, m)

    @pl.when(pl.program_id(0) == 0)
    def _():
        out_ref[...] = jnp.zeros_like(out_ref)

    ssum = jnp.sum(y, axis=0, keepdims=True)
    ssq = jnp.sum(y * y, axis=0, keepdims=True)
    out_ref[...] += jnp.concatenate([ssum, ssq], axis=0)


def _main_body(nb, m, e_total, stats_ref, ag_ref, a_ref, b3_ref, ws_ref,
               wn_ref, wb_ref, b_ref, g1_ref, b1_ref, out_ref):
    y = _edge_preact(ag_ref[...], a_ref[...], b3_ref[...],
                     ws_ref[...], wn_ref[...], wb_ref[...], b_ref[...], nb, m)
    stats = stats_ref[...]
    mean = stats[0:1, :] / e_total
    var = stats[1:2, :] / e_total - mean * mean
    inv = lax.rsqrt(var + 1e-5)
    z = (y - mean) * (inv * g1_ref[...]) + b1_ref[...]
    f = z.shape[1] // 2
    filt = jax.nn.sigmoid(z[:, :f])
    core = jax.nn.softplus(z[:, f:])
    p = (filt * core).reshape(nb, m, f)
    out_ref[...] = jnp.sum(p, axis=1)


def _final_body(a_ref, *rest):
    ps_refs = rest[:-3]
    g2_ref, b2_ref, out_ref = rest[-3:]
    x = jnp.concatenate([p[...] for p in ps_refs], axis=0)
    mean = jnp.mean(x, axis=0, keepdims=True)
    d = x - mean
    var = jnp.mean(d * d, axis=0, keepdims=True)
    z = d * lax.rsqrt(var + 1e-5) * g2_ref[...] + b2_ref[...]
    out_ref[...] = jax.nn.softplus(a_ref[...] + z)


_NB = 400


def _stats_call(ag, a_k, bf_k, ws, wn, wb, b, interpret=False):
    nk, f = a_k.shape
    e_k, hp = ag.shape
    m = e_k // nk
    two_f = ws.shape[1]
    de = bf_k.shape[1]
    r = _NB * m
    full = lambda shape: pl.BlockSpec(shape, lambda i: (0,) * len(shape))
    return pl.pallas_call(
        functools.partial(_stats_body, _NB, m),
        grid=(nk // _NB,),
        in_specs=[
            pl.BlockSpec((r, hp), lambda i: (i, 0)),
            pl.BlockSpec((_NB, f), lambda i: (i, 0)),
            pl.BlockSpec((r, de), lambda i: (i, 0)),
            full((f, two_f)), full((f, two_f)), full((de, two_f)),
            full((1, two_f)),
        ],
        out_specs=pl.BlockSpec((2, two_f), lambda i: (0, 0)),
        out_shape=jax.ShapeDtypeStruct((2, two_f), jnp.float32),
        interpret=interpret,
    )(ag, a_k, bf_k, ws, wn, wb, b)


def _main_call(stats, ag, a_k, bf_k, ws, wn, wb, b, g1, b1, e_total,
               interpret=False):
    nk, f = a_k.shape
    e_k, hp = ag.shape
    m = e_k // nk
    two_f = ws.shape[1]
    de = bf_k.shape[1]
    r = _NB * m
    full = lambda shape: pl.BlockSpec(shape, lambda i: (0,) * len(shape))
    return pl.pallas_call(
        functools.partial(_main_body, _NB, m, float(e_total)),
        grid=(nk // _NB,),
        in_specs=[
            full((2, two_f)),
            pl.BlockSpec((r, hp), lambda i: (i, 0)),
            pl.BlockSpec((_NB, f), lambda i: (i, 0)),
            pl.BlockSpec((r, de), lambda i: (i, 0)),
            full((f, two_f)), full((f, two_f)), full((de, two_f)),
            full((1, two_f)), full((1, two_f)), full((1, two_f)),
        ],
        out_specs=pl.BlockSpec((_NB, f), lambda i: (i, 0)),
        out_shape=jax.ShapeDtypeStruct((nk, f), jnp.float32),
        interpret=interpret,
    )(stats, ag, a_k, bf_k, ws, wn, wb, b, g1, b1)


_K = 1


def kernel(atom_features, bond_features, W, b, bn1_scale, bn1_bias,
           bn2_scale, bn2_bias, neighbor_indices):
    a = atom_features
    n, f = a.shape
    m = neighbor_indices.shape[1]
    idx_flat = neighbor_indices.reshape(-1)
    de = bond_features.shape[2]
    bflat = bond_features.reshape(n * m, de).astype(jnp.bfloat16)
    w16 = W.astype(jnp.bfloat16)
    ws, wn, wb = w16[:f], w16[f:2 * f], w16[2 * f:]
    b1r = b.reshape(1, -1)
    g1 = bn1_scale.reshape(1, -1)
    bb1 = bn1_bias.reshape(1, -1)
    g2 = bn2_scale.reshape(1, -1)
    bb2 = bn2_bias.reshape(1, -1)

    e_total = n * m
    ek, nk = e_total // _K, n // _K
    ags, sts = [], []
    for k in range(_K):
        ags.append(_sc_gather(a, idx_flat[k * ek:(k + 1) * ek]))
        sts.append(_stats_call(ags[k], a[k * nk:(k + 1) * nk],
                               bflat[k * ek:(k + 1) * ek], ws, wn, wb, b1r))
    stats = sts[0]
    for st in sts[1:]:
        stats = stats + st
    presums = [
        _main_call(stats, ags[k], a[k * nk:(k + 1) * nk],
                   bflat[k * ek:(k + 1) * ek], ws, wn, wb, b1r, g1, bb1,
                   e_total)
        for k in range(_K)
    ]
    out = pl.pallas_call(
        _final_body,
        out_shape=jax.ShapeDtypeStruct((n, f), jnp.float32),
    )(a, *presums, g2, bb2)
    return out


# speedup vs baseline: 1.2377x; 1.0802x over previous
import functools

import jax
import jax.numpy as jnp
from jax import lax
from jax.experimental import pallas as pl
from jax.experimental.pallas import tpu as pltpu
from jax.experimental.pallas import tpu_sc as plsc

_NC, _NS = 2, 16
_NW = _NC * _NS
_CH = 400
_SUB = 400


def _sc_gather(table, idx_flat):
    e_total = idx_flat.shape[0]
    f = table.shape[1]
    dt = table.dtype
    rows_per_w = e_total // _NW
    n_ch = rows_per_w // _CH
    n_sub = _CH // _SUB
    mesh = plsc.VectorSubcoreMesh(core_axis_name="c", subcore_axis_name="s",
                                  num_cores=_NC, num_subcores=_NS)

    @functools.partial(
        pl.kernel, mesh=mesh,
        out_type=jax.ShapeDtypeStruct((e_total, f), dt),
        scratch_types=[
            pltpu.VMEM((_CH,), jnp.int32),
            pltpu.VMEM((_CH,), jnp.int32),
            pltpu.VMEM((_CH, f), dt),
            pltpu.VMEM((_CH, f), dt),
            pltpu.SemaphoreType.DMA,
            pltpu.SemaphoreType.DMA,
            pltpu.SemaphoreType.DMA,
        ],
    )
    def k(table_hbm, idx_hbm, out_hbm, idx_v0, idx_v1, rows_v0, rows_v1,
          isem, gsem, osem):
        wid = lax.axis_index("s") * _NC + lax.axis_index("c")
        base = wid * rows_per_w
        idx_vs = (idx_v0, idx_v1)
        rows_vs = (rows_v0, rows_v1)

        def idx_cp(i, slot):
            return pltpu.make_async_copy(
                idx_hbm.at[pl.ds(base + i * _CH, _CH)], idx_vs[slot], isem)

        def gather_cps(slot):
            return [pltpu.make_async_copy(
                table_hbm.at[idx_vs[slot].at[pl.ds(j * _SUB, _SUB)]],
                rows_vs[slot].at[pl.ds(j * _SUB, _SUB)], gsem)
                for j in range(n_sub)]

        def out_cp(i, slot):
            return pltpu.make_async_copy(
                rows_vs[slot], out_hbm.at[pl.ds(base + i * _CH, _CH)], osem)

        def body(i, carry):
            idx_cp(i, 0).start()
            idx_cp(i, 0).wait()
            copies = gather_cps(0)
            for cp in copies:
                cp.start()
            for cp in copies:
                cp.wait()
            out_cp(i, 0).start()
            out_cp(i, 0).wait()
            return carry

        lax.fori_loop(0, n_ch, body, 0)

    return k(table, idx_flat)


def _edge_preact(ag, a_blk, b3, ws, wn, wb, b, nb, m):
    two_f = ws.shape[1]
    bf = jnp.bfloat16
    s = jnp.dot(a_blk.astype(bf), ws, preferred_element_type=jnp.float32)
    ynb = jnp.dot(ag.astype(bf), wn, preferred_element_type=jnp.float32)
    q = jnp.dot(b3, wb, preferred_element_type=jnp.float32)
    s_exp = jnp.broadcast_to(s[:, None, :], (nb, m, two_f)).reshape(nb * m, two_f)
    return ynb + q + s_exp + b


def _stats_body(nb, m, ag_ref, a_ref, b3_ref, ws_ref, wn_ref, wb_ref,
                b_ref, out_ref):
    y = _edge_preact(ag_ref[...], a_ref[...], b3_ref[...],
                     ws_ref[...], wn_ref[...], wb_ref[...], b_ref[...], nb, m)

    @pl.when(pl.program_id(0) == 0)
    def _():
        out_ref[...] = jnp.zeros_like(out_ref)

    ssum = jnp.sum(y, axis=0, keepdims=True)
    ssq = jnp.sum(y * y, axis=0, keepdims=True)
    out_ref[...] += jnp.concatenate([ssum, ssq], axis=0)


def _main_body(nb, m, e_total, stats_ref, ag_ref, a_ref, b3_ref, ws_ref,
               wn_ref, wb_ref, b_ref, g1_ref, b1_ref, out_ref):
    y = _edge_preact(ag_ref[...], a_ref[...], b3_ref[...],
                     ws_ref[...], wn_ref[...], wb_ref[...], b_ref[...], nb, m)
    stats = stats_ref[...]
    mean = stats[0:1, :] / e_total
    var = stats[1:2, :] / e_total - mean * mean
    inv = lax.rsqrt(var + 1e-5)
    z = (y - mean) * (inv * g1_ref[...]) + b1_ref[...]
    f = z.shape[1] // 2
    zh = z.astype(jnp.bfloat16)
    filt = jax.nn.sigmoid(zh[:, :f])
    core = jax.nn.softplus(zh[:, f:])
    p = (filt * core).astype(jnp.float32).reshape(nb, m, f)
    out_ref[...] = jnp.sum(p, axis=1)


def _final_body(a_ref, *rest):
    ps_refs = rest[:-3]
    g2_ref, b2_ref, out_ref = rest[-3:]
    x = jnp.concatenate([p[...] for p in ps_refs], axis=0)
    mean = jnp.mean(x, axis=0, keepdims=True)
    d = x - mean
    var = jnp.mean(d * d, axis=0, keepdims=True)
    z = d * lax.rsqrt(var + 1e-5) * g2_ref[...] + b2_ref[...]
    out_ref[...] = jax.nn.softplus(a_ref[...] + z)


_NB = 400


def _stats_call(ag, a_k, bf_k, ws, wn, wb, b, interpret=False):
    nk, f = a_k.shape
    e_k, hp = ag.shape
    m = e_k // nk
    two_f = ws.shape[1]
    de = bf_k.shape[1]
    r = _NB * m
    full = lambda shape: pl.BlockSpec(shape, lambda i: (0,) * len(shape))
    return pl.pallas_call(
        functools.partial(_stats_body, _NB, m),
        grid=(nk // _NB,),
        in_specs=[
            pl.BlockSpec((r, hp), lambda i: (i, 0)),
            pl.BlockSpec((_NB, f), lambda i: (i, 0)),
            pl.BlockSpec((r, de), lambda i: (i, 0)),
            full((f, two_f)), full((f, two_f)), full((de, two_f)),
            full((1, two_f)),
        ],
        out_specs=pl.BlockSpec((2, two_f), lambda i: (0, 0)),
        out_shape=jax.ShapeDtypeStruct((2, two_f), jnp.float32),
        interpret=interpret,
    )(ag, a_k, bf_k, ws, wn, wb, b)


def _main_call(stats, ag, a_k, bf_k, ws, wn, wb, b, g1, b1, e_total,
               interpret=False):
    nk, f = a_k.shape
    e_k, hp = ag.shape
    m = e_k // nk
    two_f = ws.shape[1]
    de = bf_k.shape[1]
    r = _NB * m
    full = lambda shape: pl.BlockSpec(shape, lambda i: (0,) * len(shape))
    return pl.pallas_call(
        functools.partial(_main_body, _NB, m, float(e_total)),
        grid=(nk // _NB,),
        in_specs=[
            full((2, two_f)),
            pl.BlockSpec((r, hp), lambda i: (i, 0)),
            pl.BlockSpec((_NB, f), lambda i: (i, 0)),
            pl.BlockSpec((r, de), lambda i: (i, 0)),
            full((f, two_f)), full((f, two_f)), full((de, two_f)),
            full((1, two_f)), full((1, two_f)), full((1, two_f)),
        ],
        out_specs=pl.BlockSpec((_NB, f), lambda i: (i, 0)),
        out_shape=jax.ShapeDtypeStruct((nk, f), jnp.float32),
        interpret=interpret,
    )(stats, ag, a_k, bf_k, ws, wn, wb, b, g1, b1)


_K = 1


def kernel(atom_features, bond_features, W, b, bn1_scale, bn1_bias,
           bn2_scale, bn2_bias, neighbor_indices):
    a = atom_features
    n, f = a.shape
    m = neighbor_indices.shape[1]
    idx_flat = neighbor_indices.reshape(-1)
    de = bond_features.shape[2]
    bflat = bond_features.reshape(n * m, de).astype(jnp.bfloat16)
    w16 = W.astype(jnp.bfloat16)
    ws, wn, wb = w16[:f], w16[f:2 * f], w16[2 * f:]
    b1r = b.reshape(1, -1)
    g1 = bn1_scale.reshape(1, -1)
    bb1 = bn1_bias.reshape(1, -1)
    g2 = bn2_scale.reshape(1, -1)
    bb2 = bn2_bias.reshape(1, -1)

    e_total = n * m
    ek, nk = e_total // _K, n // _K
    ags, sts = [], []
    for k in range(_K):
        ags.append(_sc_gather(a, idx_flat[k * ek:(k + 1) * ek]))
        sts.append(_stats_call(ags[k], a[k * nk:(k + 1) * nk],
                               bflat[k * ek:(k + 1) * ek], ws, wn, wb, b1r))
    stats = sts[0]
    for st in sts[1:]:
        stats = stats + st
    presums = [
        _main_call(stats, ags[k], a[k * nk:(k + 1) * nk],
                   bflat[k * ek:(k + 1) * ek], ws, wn, wb, b1r, g1, bb1,
                   e_total)
        for k in range(_K)
    ]
    out = pl.pallas_call(
        _final_body,
        out_shape=jax.ShapeDtypeStruct((n, f), jnp.float32),
    )(a, *presums, g2, bb2)
    return out


# speedup vs baseline: 1.2633x; 1.0206x over previous
import functools

import jax
import jax.numpy as jnp
from jax import lax
from jax.experimental import pallas as pl
from jax.experimental.pallas import tpu as pltpu
from jax.experimental.pallas import tpu_sc as plsc

_NC, _NS = 2, 16
_NW = _NC * _NS
_CH = 400
_SUB = 400


def _sc_gather(table, idx_flat):
    e_total = idx_flat.shape[0]
    f = table.shape[1]
    dt = table.dtype
    rows_per_w = e_total // _NW
    n_ch = rows_per_w // _CH
    n_sub = _CH // _SUB
    mesh = plsc.VectorSubcoreMesh(core_axis_name="c", subcore_axis_name="s",
                                  num_cores=_NC, num_subcores=_NS)

    @functools.partial(
        pl.kernel, mesh=mesh,
        out_type=jax.ShapeDtypeStruct((e_total, f), dt),
        scratch_types=[
            pltpu.VMEM((_CH,), jnp.int32),
            pltpu.VMEM((_CH,), jnp.int32),
            pltpu.VMEM((_CH, f), dt),
            pltpu.VMEM((_CH, f), dt),
            pltpu.SemaphoreType.DMA,
            pltpu.SemaphoreType.DMA,
            pltpu.SemaphoreType.DMA,
        ],
    )
    def k(table_hbm, idx_hbm, out_hbm, idx_v0, idx_v1, rows_v0, rows_v1,
          isem, gsem, osem):
        wid = lax.axis_index("s") * _NC + lax.axis_index("c")
        base = wid * rows_per_w
        idx_vs = (idx_v0, idx_v1)
        rows_vs = (rows_v0, rows_v1)

        def idx_cp(i, slot):
            return pltpu.make_async_copy(
                idx_hbm.at[pl.ds(base + i * _CH, _CH)], idx_vs[slot], isem)

        def gather_cps(slot):
            return [pltpu.make_async_copy(
                table_hbm.at[idx_vs[slot].at[pl.ds(j * _SUB, _SUB)]],
                rows_vs[slot].at[pl.ds(j * _SUB, _SUB)], gsem)
                for j in range(n_sub)]

        def out_cp(i, slot):
            return pltpu.make_async_copy(
                rows_vs[slot], out_hbm.at[pl.ds(base + i * _CH, _CH)], osem)

        def body(i, carry):
            idx_cp(i, 0).start()
            idx_cp(i, 0).wait()
            copies = gather_cps(0)
            for cp in copies:
                cp.start()
            for cp in copies:
                cp.wait()
            out_cp(i, 0).start()
            out_cp(i, 0).wait()
            return carry

        lax.fori_loop(0, n_ch, body, 0)

    return k(table, idx_flat)


def _edge_preact(ag, a_blk, b3, ws, wn, wb, b, nb, m, dt=jnp.float32):
    two_f = ws.shape[1]
    bf = jnp.bfloat16
    s = jnp.dot(a_blk.astype(bf), ws, preferred_element_type=dt)
    ynb = jnp.dot(ag.astype(bf), wn, preferred_element_type=dt)
    q = jnp.dot(b3, wb, preferred_element_type=dt)
    s_exp = jnp.broadcast_to(s[:, None, :], (nb, m, two_f)).reshape(nb * m, two_f)
    return ynb + q + s_exp + b.astype(dt)


def _stats_body(nb, m, ag_ref, a_ref, b3_ref, ws_ref, wn_ref, wb_ref,
                b_ref, out_ref):
    y = _edge_preact(ag_ref[...], a_ref[...], b3_ref[...],
                     ws_ref[...], wn_ref[...], wb_ref[...], b_ref[...], nb, m)

    @pl.when(pl.program_id(0) == 0)
    def _():
        out_ref[...] = jnp.zeros_like(out_ref)

    ssum = jnp.sum(y, axis=0, keepdims=True)
    ssq = jnp.sum(y * y, axis=0, keepdims=True)
    out_ref[...] += jnp.concatenate([ssum, ssq], axis=0)


def _main_body(nb, m, e_total, stats_ref, ag_ref, a_ref, b3_ref, ws_ref,
               wn_ref, wb_ref, b_ref, g1_ref, b1_ref, out_ref):
    bf = jnp.bfloat16
    y = _edge_preact(ag_ref[...], a_ref[...], b3_ref[...],
                     ws_ref[...], wn_ref[...], wb_ref[...], b_ref[...], nb, m)
    stats = stats_ref[...]
    mean = stats[0:1, :] / e_total
    var = stats[1:2, :] / e_total - mean * mean
    inv = lax.rsqrt(var + 1e-5)
    s1 = inv * g1_ref[...]
    t1 = b1_ref[...] - mean * s1
    z = (y * s1 + t1).astype(bf)
    f = z.shape[1] // 2
    filt = jax.nn.sigmoid(z[:, :f])
    core = jax.nn.softplus(z[:, f:])
    p = (filt * core).astype(jnp.float32).reshape(nb, m, f)
    out_ref[...] = jnp.sum(p, axis=1)


def _final_body(a_ref, *rest):
    ps_refs = rest[:-3]
    g2_ref, b2_ref, out_ref = rest[-3:]
    x = jnp.concatenate([p[...] for p in ps_refs], axis=0)
    mean = jnp.mean(x, axis=0, keepdims=True)
    d = x - mean
    var = jnp.mean(d * d, axis=0, keepdims=True)
    z = d * lax.rsqrt(var + 1e-5) * g2_ref[...] + b2_ref[...]
    out_ref[...] = jax.nn.softplus(a_ref[...] + z)


_NB = 400


def _stats_call(ag, a_k, bf_k, ws, wn, wb, b, interpret=False):
    nk, f = a_k.shape
    e_k, hp = ag.shape
    m = e_k // nk
    two_f = ws.shape[1]
    de = bf_k.shape[1]
    r = _NB * m
    full = lambda shape: pl.BlockSpec(shape, lambda i: (0,) * len(shape))
    return pl.pallas_call(
        functools.partial(_stats_body, _NB, m),
        grid=(nk // _NB,),
        in_specs=[
            pl.BlockSpec((r, hp), lambda i: (i, 0)),
            pl.BlockSpec((_NB, f), lambda i: (i, 0)),
            pl.BlockSpec((r, de), lambda i: (i, 0)),
            full((f, two_f)), full((f, two_f)), full((de, two_f)),
            full((1, two_f)),
        ],
        out_specs=pl.BlockSpec((2, two_f), lambda i: (0, 0)),
        out_shape=jax.ShapeDtypeStruct((2, two_f), jnp.float32),
        interpret=interpret,
    )(ag, a_k, bf_k, ws, wn, wb, b)


def _main_call(stats, ag, a_k, bf_k, ws, wn, wb, b, g1, b1, e_total,
               interpret=False):
    nk, f = a_k.shape
    e_k, hp = ag.shape
    m = e_k // nk
    two_f = ws.shape[1]
    de = bf_k.shape[1]
    r = _NB * m
    full = lambda shape: pl.BlockSpec(shape, lambda i: (0,) * len(shape))
    return pl.pallas_call(
        functools.partial(_main_body, _NB, m, float(e_total)),
        grid=(nk // _NB,),
        in_specs=[
            full((2, two_f)),
            pl.BlockSpec((r, hp), lambda i: (i, 0)),
            pl.BlockSpec((_NB, f), lambda i: (i, 0)),
            pl.BlockSpec((r, de), lambda i: (i, 0)),
            full((f, two_f)), full((f, two_f)), full((de, two_f)),
            full((1, two_f)), full((1, two_f)), full((1, two_f)),
        ],
        out_specs=pl.BlockSpec((_NB, f), lambda i: (i, 0)),
        out_shape=jax.ShapeDtypeStruct((nk, f), jnp.float32),
        interpret=interpret,
    )(stats, ag, a_k, bf_k, ws, wn, wb, b, g1, b1)


_K = 1


def kernel(atom_features, bond_features, W, b, bn1_scale, bn1_bias,
           bn2_scale, bn2_bias, neighbor_indices):
    a = atom_features
    n, f = a.shape
    m = neighbor_indices.shape[1]
    idx_flat = neighbor_indices.reshape(-1)
    de = bond_features.shape[2]
    bflat = bond_features.reshape(n * m, de).astype(jnp.bfloat16)
    w16 = W.astype(jnp.bfloat16)
    ws, wn, wb = w16[:f], w16[f:2 * f], w16[2 * f:]
    b1r = b.reshape(1, -1)
    g1 = bn1_scale.reshape(1, -1)
    bb1 = bn1_bias.reshape(1, -1)
    g2 = bn2_scale.reshape(1, -1)
    bb2 = bn2_bias.reshape(1, -1)

    e_total = n * m
    ek, nk = e_total // _K, n // _K
    ags, sts = [], []
    for k in range(_K):
        ags.append(_sc_gather(a, idx_flat[k * ek:(k + 1) * ek]))
        sts.append(_stats_call(ags[k], a[k * nk:(k + 1) * nk],
                               bflat[k * ek:(k + 1) * ek], ws, wn, wb, b1r))
    stats = sts[0]
    for st in sts[1:]:
        stats = stats + st
    presums = [
        _main_call(stats, ags[k], a[k * nk:(k + 1) * nk],
                   bflat[k * ek:(k + 1) * ek], ws, wn, wb, b1r, g1, bb1,
                   e_total)
        for k in range(_K)
    ]
    out = pl.pallas_call(
        _final_body,
        out_shape=jax.ShapeDtypeStruct((n, f), jnp.float32),
    )(a, *presums, g2, bb2)
    return out


# speedup vs baseline: 1.3157x; 1.0415x over previous
import functools

import jax
import jax.numpy as jnp
from jax import lax
from jax.experimental import pallas as pl
from jax.experimental.pallas import tpu as pltpu
from jax.experimental.pallas import tpu_sc as plsc

_NC, _NS = 2, 16
_NW = _NC * _NS
_CH = 1000
_SUB = 1000


def _sc_gather(table, idx_flat):
    e_total = idx_flat.shape[0]
    f = table.shape[1]
    dt = table.dtype
    rows_per_w = e_total // _NW
    n_ch = rows_per_w // _CH
    n_sub = _CH // _SUB
    mesh = plsc.VectorSubcoreMesh(core_axis_name="c", subcore_axis_name="s",
                                  num_cores=_NC, num_subcores=_NS)

    @functools.partial(
        pl.kernel, mesh=mesh,
        out_type=jax.ShapeDtypeStruct((e_total, f), dt),
        scratch_types=[
            pltpu.VMEM((_CH,), jnp.int32),
            pltpu.VMEM((_CH, f), dt),
            pltpu.SemaphoreType.DMA,
            pltpu.SemaphoreType.DMA,
            pltpu.SemaphoreType.DMA,
        ],
    )
    def k(table_hbm, idx_hbm, out_hbm, idx_v0, rows_v0,
          isem, gsem, osem):
        wid = lax.axis_index("s") * _NC + lax.axis_index("c")
        base = wid * rows_per_w
        idx_vs = (idx_v0,)
        rows_vs = (rows_v0,)

        def idx_cp(i, slot):
            return pltpu.make_async_copy(
                idx_hbm.at[pl.ds(base + i * _CH, _CH)], idx_vs[slot], isem)

        def gather_cps(slot):
            return [pltpu.make_async_copy(
                table_hbm.at[idx_vs[slot].at[pl.ds(j * _SUB, _SUB)]],
                rows_vs[slot].at[pl.ds(j * _SUB, _SUB)], gsem)
                for j in range(n_sub)]

        def out_cp(i, slot):
            return pltpu.make_async_copy(
                rows_vs[slot], out_hbm.at[pl.ds(base + i * _CH, _CH)], osem)

        def body(i, carry):
            idx_cp(i, 0).start()
            idx_cp(i, 0).wait()
            copies = gather_cps(0)
            for cp in copies:
                cp.start()
            for cp in copies:
                cp.wait()
            out_cp(i, 0).start()
            out_cp(i, 0).wait()
            return carry

        lax.fori_loop(0, n_ch, body, 0)

    return k(table, idx_flat)


def _edge_preact(ag, a_blk, b3, ws, wn, wb, b, nb, m, dt=jnp.float32):
    two_f = ws.shape[1]
    bf = jnp.bfloat16
    s = jnp.dot(a_blk.astype(bf), ws, preferred_element_type=dt)
    ynb = jnp.dot(ag.astype(bf), wn, preferred_element_type=dt)
    q = jnp.dot(b3, wb, preferred_element_type=dt)
    s_exp = jnp.broadcast_to(s[:, None, :], (nb, m, two_f)).reshape(nb * m, two_f)
    return ynb + q + s_exp + b.astype(dt)


def _stats_body(nb, m, ag_ref, a_ref, b3_ref, ws_ref, wn_ref, wb_ref,
                b_ref, out_ref):
    y = _edge_preact(ag_ref[...], a_ref[...], b3_ref[...],
                     ws_ref[...], wn_ref[...], wb_ref[...], b_ref[...], nb, m)

    @pl.when(pl.program_id(0) == 0)
    def _():
        out_ref[...] = jnp.zeros_like(out_ref)

    ssum = jnp.sum(y, axis=0, keepdims=True)
    ssq = jnp.sum(y * y, axis=0, keepdims=True)
    out_ref[...] += jnp.concatenate([ssum, ssq], axis=0)


def _main_body(nb, m, e_total, stats_ref, ag_ref, a_ref, b3_ref, ws_ref,
               wn_ref, wb_ref, b_ref, g1_ref, b1_ref, out_ref):
    bf = jnp.bfloat16
    y = _edge_preact(ag_ref[...], a_ref[...], b3_ref[...],
                     ws_ref[...], wn_ref[...], wb_ref[...], b_ref[...], nb, m)
    stats = stats_ref[...]
    mean = stats[0:1, :] / e_total
    var = stats[1:2, :] / e_total - mean * mean
    inv = lax.rsqrt(var + 1e-5)
    s1 = inv * g1_ref[...]
    t1 = b1_ref[...] - mean * s1
    z = (y * s1 + t1).astype(bf)
    f = z.shape[1] // 2
    filt = jax.nn.sigmoid(z[:, :f])
    core = jax.nn.softplus(z[:, f:])
    p = (filt * core).astype(jnp.float32).reshape(nb, m, f)
    out_ref[...] = jnp.sum(p, axis=1)


def _final_body(a_ref, *rest):
    ps_refs = rest[:-3]
    g2_ref, b2_ref, out_ref = rest[-3:]
    x = jnp.concatenate([p[...] for p in ps_refs], axis=0)
    mean = jnp.mean(x, axis=0, keepdims=True)
    d = x - mean
    var = jnp.mean(d * d, axis=0, keepdims=True)
    z = d * lax.rsqrt(var + 1e-5) * g2_ref[...] + b2_ref[...]
    out_ref[...] = jax.nn.softplus(a_ref[...] + z)


_NB = 400


def _stats_call(ag, a_k, bf_k, ws, wn, wb, b, interpret=False):
    nk, f = a_k.shape
    e_k, hp = ag.shape
    m = e_k // nk
    two_f = ws.shape[1]
    de = bf_k.shape[1]
    r = _NB * m
    full = lambda shape: pl.BlockSpec(shape, lambda i: (0,) * len(shape))
    return pl.pallas_call(
        functools.partial(_stats_body, _NB, m),
        grid=(nk // _NB,),
        in_specs=[
            pl.BlockSpec((r, hp), lambda i: (i, 0)),
            pl.BlockSpec((_NB, f), lambda i: (i, 0)),
            pl.BlockSpec((r, de), lambda i: (i, 0)),
            full((f, two_f)), full((f, two_f)), full((de, two_f)),
            full((1, two_f)),
        ],
        out_specs=pl.BlockSpec((2, two_f), lambda i: (0, 0)),
        out_shape=jax.ShapeDtypeStruct((2, two_f), jnp.float32),
        interpret=interpret,
    )(ag, a_k, bf_k, ws, wn, wb, b)


def _main_call(stats, ag, a_k, bf_k, ws, wn, wb, b, g1, b1, e_total,
               interpret=False):
    nk, f = a_k.shape
    e_k, hp = ag.shape
    m = e_k // nk
    two_f = ws.shape[1]
    de = bf_k.shape[1]
    r = _NB * m
    full = lambda shape: pl.BlockSpec(shape, lambda i: (0,) * len(shape))
    return pl.pallas_call(
        functools.partial(_main_body, _NB, m, float(e_total)),
        grid=(nk // _NB,),
        in_specs=[
            full((2, two_f)),
            pl.BlockSpec((r, hp), lambda i: (i, 0)),
            pl.BlockSpec((_NB, f), lambda i: (i, 0)),
            pl.BlockSpec((r, de), lambda i: (i, 0)),
            full((f, two_f)), full((f, two_f)), full((de, two_f)),
            full((1, two_f)), full((1, two_f)), full((1, two_f)),
        ],
        out_specs=pl.BlockSpec((_NB, f), lambda i: (i, 0)),
        out_shape=jax.ShapeDtypeStruct((nk, f), jnp.float32),
        interpret=interpret,
    )(stats, ag, a_k, bf_k, ws, wn, wb, b, g1, b1)


_K = 1


def kernel(atom_features, bond_features, W, b, bn1_scale, bn1_bias,
           bn2_scale, bn2_bias, neighbor_indices):
    a = atom_features
    n, f = a.shape
    m = neighbor_indices.shape[1]
    idx_flat = neighbor_indices.reshape(-1)
    de = bond_features.shape[2]
    bflat = bond_features.reshape(n * m, de).astype(jnp.bfloat16)
    w16 = W.astype(jnp.bfloat16)
    ws, wn, wb = w16[:f], w16[f:2 * f], w16[2 * f:]
    b1r = b.reshape(1, -1)
    g1 = bn1_scale.reshape(1, -1)
    bb1 = bn1_bias.reshape(1, -1)
    g2 = bn2_scale.reshape(1, -1)
    bb2 = bn2_bias.reshape(1, -1)

    e_total = n * m
    ek, nk = e_total // _K, n // _K
    ags, sts = [], []
    for k in range(_K):
        ags.append(_sc_gather(a, idx_flat[k * ek:(k + 1) * ek]))
        sts.append(_stats_call(ags[k], a[k * nk:(k + 1) * nk],
                               bflat[k * ek:(k + 1) * ek], ws, wn, wb, b1r))
    stats = sts[0]
    for st in sts[1:]:
        stats = stats + st
    presums = [
        _main_call(stats, ags[k], a[k * nk:(k + 1) * nk],
                   bflat[k * ek:(k + 1) * ek], ws, wn, wb, b1r, g1, bb1,
                   e_total)
        for k in range(_K)
    ]
    out = pl.pallas_call(
        _final_body,
        out_shape=jax.ShapeDtypeStruct((n, f), jnp.float32),
    )(a, *presums, g2, bb2)
    return out
